# baseline (speedup 1.0000x reference)
import jax
import jax.numpy as jnp
from jax.experimental import pallas as pl

BATCH = 16384
NUM_FEATURES = 16384
NUM_CLASSES = 2

BM = 128
BK = 8192
NK = NUM_FEATURES // BK


def _linear_kernel(x_ref, w_ref, b_ref, o_ref):
    k = pl.program_id(1)
    acc = jax.lax.dot_general(
        x_ref[...], w_ref[...], (((1,), (1,)), ((), ())),
        preferred_element_type=jnp.float32,
    )

    @pl.when(k == 0)
    def _():
        o_ref[...] = acc + b_ref[...]

    @pl.when(k != 0)
    def _():
        o_ref[...] += acc


def kernel(batch, W, b):
    b2 = b.reshape(1, NUM_CLASSES)
    return pl.pallas_call(
        _linear_kernel,
        grid=(BATCH // BM, NK),
        in_specs=[
            pl.BlockSpec((BM, BK), lambda i, k: (i, k)),
            pl.BlockSpec((NUM_CLASSES, BK), lambda i, k: (0, k)),
            pl.BlockSpec((1, NUM_CLASSES), lambda i, k: (0, 0)),
        ],
        out_specs=pl.BlockSpec((BM, NUM_CLASSES), lambda i, k: (i, 0)),
        out_shape=jax.ShapeDtypeStruct((BATCH, NUM_CLASSES), jnp.float32),
    )(batch, W, b2)


# speedup vs baseline: 1.2682x; 1.2682x over previous
import jax
import jax.numpy as jnp
from jax.experimental import pallas as pl
from jax.experimental.pallas import tpu as pltpu

BATCH = 16384
NUM_FEATURES = 16384
NUM_CLASSES = 2

BM = 128


def _linear_kernel(x_ref, w_ref, b_ref, o_ref):
    acc = jax.lax.dot_general(
        x_ref[...], w_ref[...], (((1,), (1,)), ((), ())),
        preferred_element_type=jnp.float32,
    )
    o_ref[...] = acc + b_ref[...]


def kernel(batch, W, b):
    b2 = b.reshape(1, NUM_CLASSES)
    return pl.pallas_call(
        _linear_kernel,
        grid=(BATCH // BM,),
        in_specs=[
            pl.BlockSpec((BM, NUM_FEATURES), lambda i: (i, 0)),
            pl.BlockSpec((NUM_CLASSES, NUM_FEATURES), lambda i: (0, 0)),
            pl.BlockSpec((1, NUM_CLASSES), lambda i: (0, 0)),
        ],
        out_specs=pl.BlockSpec((BM, NUM_CLASSES), lambda i: (i, 0)),
        out_shape=jax.ShapeDtypeStruct((BATCH, NUM_CLASSES), jnp.float32),
        compiler_params=pltpu.CompilerParams(
            dimension_semantics=("parallel",),
        ),
    )(batch, W, b2)


# speedup vs baseline: 1.2713x; 1.0024x over previous
import jax
import jax.numpy as jnp
from jax.experimental import pallas as pl
from jax.experimental.pallas import tpu as pltpu

BATCH = 16384
NUM_FEATURES = 16384
NUM_CLASSES = 2

BM = 256


def _linear_kernel(x_ref, w_ref, b_ref, o_ref):
    acc = jax.lax.dot_general(
        x_ref[...], w_ref[...], (((1,), (1,)), ((), ())),
        preferred_element_type=jnp.float32,
    )
    o_ref[...] = acc + b_ref[...]


def kernel(batch, W, b):
    b2 = b.reshape(1, NUM_CLASSES)
    return pl.pallas_call(
        _linear_kernel,
        grid=(BATCH // BM,),
        in_specs=[
            pl.BlockSpec((BM, NUM_FEATURES), lambda i: (i, 0)),
            pl.BlockSpec((NUM_CLASSES, NUM_FEATURES), lambda i: (0, 0)),
            pl.BlockSpec((1, NUM_CLASSES), lambda i: (0, 0)),
        ],
        out_specs=pl.BlockSpec((BM, NUM_CLASSES), lambda i: (i, 0)),
        out_shape=jax.ShapeDtypeStruct((BATCH, NUM_CLASSES), jnp.float32),
        compiler_params=pltpu.CompilerParams(
            dimension_semantics=("parallel",),
            vmem_limit_bytes=60 * 1024 * 1024,
        ),
    )(batch, W, b2)
